# baseline (speedup 1.0000x reference)
import functools

import jax
import jax.numpy as jnp
from jax import lax
from jax.experimental import pallas as pl
from jax.experimental.pallas import tpu as pltpu
from jax.experimental.pallas import tpu_sc as plsc


@functools.lru_cache(maxsize=None)
def _build_sc_transpose(N, D):
    info = plsc.get_sparse_core_info()
    NC, NS, L = info.num_cores, info.num_subcores, info.num_lanes
    NW = NC * NS
    CT = 128
    CD = CT * D
    NCH = N // CT
    KMAX = -(-NCH // NW)

    mesh = plsc.VectorSubcoreMesh(core_axis_name="c", subcore_axis_name="s")

    @functools.partial(
        pl.kernel,
        out_type=jax.ShapeDtypeStruct((N * D,), jnp.float32),
        mesh=mesh,
        compiler_params=pltpu.CompilerParams(
            needs_layout_passes=False, use_tc_tiling_on_sc=True),
        scratch_types=[
            pltpu.VMEM((4 * D, CT), jnp.float32),
            pltpu.VMEM((4 * CD,), jnp.float32),
            pltpu.SemaphoreType.DMA,
            pltpu.SemaphoreType.DMA,
            pltpu.SemaphoreType.DMA,
            pltpu.SemaphoreType.DMA,
            pltpu.SemaphoreType.DMA,
            pltpu.SemaphoreType.DMA,
            pltpu.SemaphoreType.DMA,
            pltpu.SemaphoreType.DMA,
        ],
    )
    def sc_transpose(embT_hbm, tail_hbm, out_hbm, inb, outb,
                     si0, si1, si2, si3, so0, so1, so2, so3):
        wid = lax.axis_index("s") * NC + lax.axis_index("c")
        sis = (si0, si1, si2, si3)
        sos = (so0, so1, so2, so3)
        nk = (NCH - wid + NW - 1) // NW

        def chunk_off(k):
            return (wid + k * NW) * CT

        def start_in(k, par):
            pltpu.async_copy(embT_hbm.at[:, pl.ds(chunk_off(k), CT)],
                             inb.at[pl.ds(par * D, D)], sis[par])

        def wait_in(par):
            pltpu.make_async_copy(embT_hbm.at[:, pl.ds(0, CT)],
                                  inb.at[pl.ds(par * D, D)], sis[par]).wait()

        def start_out(k, par):
            pltpu.async_copy(outb.at[pl.ds(par * CD, CD)],
                             out_hbm.at[pl.ds(chunk_off(k) * D, CD)],
                             sos[par])

        def wait_out(par):
            pltpu.make_async_copy(outb.at[pl.ds(par * CD, CD)],
                                  out_hbm.at[pl.ds(0, CD)], sos[par]).wait()

        lane = lax.iota(jnp.int32, L)
        laneD = lane * D
        diag = [(lane + t) & (L - 1) for t in range(L)]

        def transpose_block(ioff, ooff):
            @plsc.parallel_loop(0, CT, step=L, unroll=1)
            def _(i0):
                ivec = i0 + lane
                for dc in range(0, D, L):
                    dc_off = ioff + dc
                    obase = ooff + i0 * D + dc
                    for t in range(L):
                        v = plsc.load_gather(inb, [dc_off + diag[t], ivec])
                        plsc.store_scatter(outb, [obase + laneD + diag[t]], v)

        start_in(0, 0)
        start_in(1, 1)
        start_in(2, 2)

        @pl.loop(0, KMAX)
        def _(k):
            par = k & 3

            @pl.when(k < nk)
            def _():
                for v in range(4):
                    @pl.when((par == v) & (k + 3 < nk))
                    def _(v=v):
                        start_in(k + 3, (v + 3) & 3)

                for v in range(4):
                    @pl.when(par == v)
                    def _(v=v):
                        wait_in(v)

                @pl.when(k >= 4)
                def _():
                    for v in range(4):
                        @pl.when(par == v)
                        def _(v=v):
                            wait_out(v)

                transpose_block(par * D, par * CD)

                for v in range(4):
                    @pl.when(par == v)
                    def _(v=v):
                        start_out(k, v)

        for v in range(4):
            wait_out(v)

        if N % CT != 0:
            @pl.when(wid == 0)
            def _():
                pltpu.sync_copy(tail_hbm, inb.at[pl.ds(0, D)])
                transpose_block(0, 0)
                pltpu.sync_copy(outb.at[pl.ds(0, CD)],
                                out_hbm.at[pl.ds((N - CT) * D, CD)])

    return sc_transpose


@functools.lru_cache(maxsize=None)
def _build_sc_scores(B, K, D, N):
    info = plsc.get_sparse_core_info()
    NC, NS, L = info.num_cores, info.num_subcores, info.num_lanes
    NW = NC * NS
    P = K + 1
    C = K + 2
    BW = B // NW
    NSUB = BW // L
    R = L * C
    DC = D // L

    assert B % NW == 0 and BW % L == 0 and D % L == 0
    assert (BW * C) % 8 == 0 and R % 8 == 0

    mesh = plsc.VectorSubcoreMesh(core_axis_name="c", subcore_axis_name="s")

    @functools.partial(
        pl.kernel,
        out_type=jax.ShapeDtypeStruct((NW, P, BW), jnp.float32),
        mesh=mesh,
        compiler_params=pltpu.CompilerParams(
            needs_layout_passes=False, use_tc_tiling_on_sc=False),
        scratch_types=[
            pltpu.VMEM((C, BW), jnp.int32),
            pltpu.VMEM((NSUB * R,), jnp.int32),
            pltpu.VMEM((4 * R, D), jnp.float32),
            pltpu.VMEM((P, BW), jnp.float32),
            pltpu.SemaphoreType.DMA,
            pltpu.SemaphoreType.DMA,
            pltpu.SemaphoreType.DMA,
            pltpu.SemaphoreType.DMA,
        ],
    )
    def sc_scores(tgt_hbm, ctx_hbm, negflat_hbm, emb_hbm, out_hbm,
                  idx_cm, idx_g, rows_v, score_v, sem0, sem1, sem2, sem3):
        wid = lax.axis_index("s") * NC + lax.axis_index("c")
        base = wid * BW
        copies = [
            pltpu.async_copy(tgt_hbm.at[pl.ds(base, BW)], idx_cm.at[0], sem0),
            pltpu.async_copy(ctx_hbm.at[pl.ds(base, BW)], idx_cm.at[1], sem0),
        ]
        for k in range(K):
            copies.append(pltpu.async_copy(
                negflat_hbm.at[pl.ds(k * B + base, BW)],
                idx_cm.at[2 + k], sem0))
        for c in copies:
            c.wait()

        @pl.loop(0, NSUB)
        def _(s):
            for c in range(C):
                idx_g[pl.ds(s * R + c * L, L)] = idx_cm[c, pl.ds(s * L, L)]

        sems = (sem0, sem1, sem2, sem3)

        def start(s, par):
            pltpu.async_copy(emb_hbm.at[idx_g.at[pl.ds(s * R, R)]],
                             rows_v.at[pl.ds(par * R, R)], sems[par])

        def wait(par):
            pltpu.make_async_copy(emb_hbm.at[idx_g.at[pl.ds(0, R)]],
                                  rows_v.at[pl.ds(par * R, R)],
                                  sems[par]).wait()

        start(0, 0)
        start(1, 1)
        start(2, 2)

        lane = lax.iota(jnp.int32, L)
        zero = jnp.zeros((L,), jnp.float32)

        @pl.loop(0, NSUB)
        def _(j):
            for p in range(P):
                score_v[p, pl.ds(j * L, L)] = zero

        def compute(s, par):
            boff = par * R
            trow = lane + boff

            @pl.loop(0, DC)
            def _(dc):
                dbase = dc * L
                cvs = [dbase + ((lane + d) & (L - 1)) for d in range(L)]
                tcols = [plsc.load_gather(rows_v, [trow, cvs[d]])
                         for d in range(L)]
                for p in range(P):
                    prow = trow + (p + 1) * L
                    acc = tcols[0] * plsc.load_gather(rows_v, [prow, cvs[0]])
                    for d in range(1, L):
                        pc = plsc.load_gather(rows_v, [prow, cvs[d]])
                        acc = acc + tcols[d] * pc
                    plsc.addupdate(score_v.at[p, pl.ds(s * L, L)], acc)

        @pl.loop(0, NSUB)
        def _(s):
            par = s & 3

            for v in range(4):
                @pl.when((par == v) & (s + 3 < NSUB))
                def _(v=v):
                    start(s + 3, (v + 3) & 3)

            for v in range(4):
                @pl.when(par == v)
                def _(v=v):
                    wait(v)

            compute(s, par)

        pltpu.sync_copy(score_v, out_hbm.at[wid])

    return sc_scores


def _tc_flatten(negT):
    M, Nc = negT.shape

    def body(i_ref, o_ref):
        o_ref[...] = i_ref[...].reshape(M * Nc // 128, 128)

    out = pl.pallas_call(
        body,
        out_shape=jax.ShapeDtypeStruct((M * Nc // 128, 128), jnp.int32),
    )(negT)
    return out.reshape(-1)


def _tc_loss(scores_flat, B, K, P):
    M, Nc = scores_flat.shape

    def body(s_ref, o_ref):
        s = s_ref[...]
        row = lax.broadcasted_iota(jnp.int32, (M, Nc), 0)
        is_pos = (row % P) == 0
        x = jnp.where(is_pos, s, -s)
        ls = jnp.log(1.0 / (1.0 + jnp.exp(-x)) + 1e-15)
        pos_sum = jnp.sum(jnp.where(is_pos, ls, 0.0))
        neg_sum = jnp.sum(ls) - pos_sum
        o_ref[0, 0] = -(pos_sum / B) - (neg_sum / (B * K))

    out = pl.pallas_call(
        body,
        out_shape=jax.ShapeDtypeStruct((1, 1), jnp.float32),
        out_specs=pl.BlockSpec(memory_space=pltpu.SMEM),
    )(scores_flat)
    return out[0, 0]


def kernel(target, context, neg_context, emb):
    B, = target.shape
    K = neg_context.shape[1]
    N, D = emb.shape
    P = K + 1

    negT = neg_context.T.astype(jnp.int32)
    negflat = _tc_flatten(negT)
    embT = emb.T
    tailT = lax.slice(embT, (0, N - 128), (D, N))
    emb_rm = _build_sc_transpose(N, D)(embT, tailT).reshape(N, D)
    scores = _build_sc_scores(B, K, D, N)(
        target.astype(jnp.int32), context.astype(jnp.int32), negflat, emb_rm)
    NW = scores.shape[0]
    return _tc_loss(scores.reshape(NW * P, B // NW), B, K, P)


# speedup vs baseline: 1.0135x; 1.0135x over previous
import functools

import jax
import jax.numpy as jnp
from jax import lax
from jax.experimental import pallas as pl
from jax.experimental.pallas import tpu as pltpu
from jax.experimental.pallas import tpu_sc as plsc


@functools.lru_cache(maxsize=None)
def _build_sc_transpose(N, D):
    info = plsc.get_sparse_core_info()
    NC, NS, L = info.num_cores, info.num_subcores, info.num_lanes
    NW = NC * NS
    CT = 128
    CD = CT * D
    NCH = N // CT
    KMAX = -(-NCH // NW)

    mesh = plsc.VectorSubcoreMesh(core_axis_name="c", subcore_axis_name="s")

    @functools.partial(
        pl.kernel,
        out_type=jax.ShapeDtypeStruct((N * D,), jnp.float32),
        mesh=mesh,
        compiler_params=pltpu.CompilerParams(
            needs_layout_passes=False, use_tc_tiling_on_sc=True),
        scratch_types=[
            pltpu.VMEM((4 * D, CT), jnp.float32),
            pltpu.VMEM((4 * CD,), jnp.float32),
            pltpu.SemaphoreType.DMA,
            pltpu.SemaphoreType.DMA,
            pltpu.SemaphoreType.DMA,
            pltpu.SemaphoreType.DMA,
            pltpu.SemaphoreType.DMA,
            pltpu.SemaphoreType.DMA,
            pltpu.SemaphoreType.DMA,
            pltpu.SemaphoreType.DMA,
        ],
    )
    def sc_transpose(embT_hbm, tail_hbm, out_hbm, inb, outb,
                     si0, si1, si2, si3, so0, so1, so2, so3):
        wid = lax.axis_index("s") * NC + lax.axis_index("c")
        sis = (si0, si1, si2, si3)
        sos = (so0, so1, so2, so3)
        nk = (NCH - wid + NW - 1) // NW

        def chunk_off(k):
            return (wid + k * NW) * CT

        def start_in(k, par):
            pltpu.async_copy(embT_hbm.at[:, pl.ds(chunk_off(k), CT)],
                             inb.at[pl.ds(par * D, D)], sis[par])

        def wait_in(par):
            pltpu.make_async_copy(embT_hbm.at[:, pl.ds(0, CT)],
                                  inb.at[pl.ds(par * D, D)], sis[par]).wait()

        def start_out(k, par):
            pltpu.async_copy(outb.at[pl.ds(par * CD, CD)],
                             out_hbm.at[pl.ds(chunk_off(k) * D, CD)],
                             sos[par])

        def wait_out(par):
            pltpu.make_async_copy(outb.at[pl.ds(par * CD, CD)],
                                  out_hbm.at[pl.ds(0, CD)], sos[par]).wait()

        lane = lax.iota(jnp.int32, L)
        laneD = lane * D
        diag = [(lane + t) & (L - 1) for t in range(L)]

        def transpose_block(ioff, ooff):
            @plsc.parallel_loop(0, CT, step=L, unroll=1)
            def _(i0):
                ivec = i0 + lane
                for dc in range(0, D, L):
                    dc_off = ioff + dc
                    obase = ooff + i0 * D + dc
                    for t in range(L):
                        v = plsc.load_gather(inb, [dc_off + diag[t], ivec])
                        plsc.store_scatter(outb, [obase + laneD + diag[t]], v)

        start_in(0, 0)
        start_in(1, 1)
        start_in(2, 2)

        @pl.loop(0, KMAX)
        def _(k):
            par = k & 3

            @pl.when(k < nk)
            def _():
                for v in range(4):
                    @pl.when((par == v) & (k + 3 < nk))
                    def _(v=v):
                        start_in(k + 3, (v + 3) & 3)

                for v in range(4):
                    @pl.when(par == v)
                    def _(v=v):
                        wait_in(v)

                @pl.when(k >= 4)
                def _():
                    for v in range(4):
                        @pl.when(par == v)
                        def _(v=v):
                            wait_out(v)

                transpose_block(par * D, par * CD)

                for v in range(4):
                    @pl.when(par == v)
                    def _(v=v):
                        start_out(k, v)

        for v in range(4):
            wait_out(v)

        if N % CT != 0:
            @pl.when(wid == 0)
            def _():
                pltpu.sync_copy(tail_hbm, inb.at[pl.ds(0, D)])
                transpose_block(0, 0)
                pltpu.sync_copy(outb.at[pl.ds(0, CD)],
                                out_hbm.at[pl.ds((N - CT) * D, CD)])

    return sc_transpose


@functools.lru_cache(maxsize=None)
def _build_sc_scores(B, K, D, N):
    info = plsc.get_sparse_core_info()
    NC, NS, L = info.num_cores, info.num_subcores, info.num_lanes
    NW = NC * NS
    P = K + 1
    C = K + 2
    BW = B // NW
    NSUB = BW // L
    R = L * C
    DC = D // L

    assert B % NW == 0 and BW % L == 0 and D % L == 0
    assert (BW * C) % 8 == 0 and R % 8 == 0

    mesh = plsc.VectorSubcoreMesh(core_axis_name="c", subcore_axis_name="s")

    @functools.partial(
        pl.kernel,
        out_type=jax.ShapeDtypeStruct((NW, P, BW), jnp.float32),
        mesh=mesh,
        compiler_params=pltpu.CompilerParams(
            needs_layout_passes=False, use_tc_tiling_on_sc=False),
        scratch_types=[
            pltpu.VMEM((C, BW), jnp.int32),
            pltpu.VMEM((NSUB * R,), jnp.int32),
            pltpu.VMEM((2 * R, D), jnp.float32),
            pltpu.VMEM((P, BW), jnp.float32),
            pltpu.SemaphoreType.DMA,
            pltpu.SemaphoreType.DMA,
        ],
    )
    def sc_scores(tgt_hbm, ctx_hbm, negflat_hbm, emb_hbm, out_hbm,
                  idx_cm, idx_g, rows_v, score_v, sem0, sem1):
        wid = lax.axis_index("s") * NC + lax.axis_index("c")
        base = wid * BW
        copies = [
            pltpu.async_copy(tgt_hbm.at[pl.ds(base, BW)], idx_cm.at[0], sem0),
            pltpu.async_copy(ctx_hbm.at[pl.ds(base, BW)], idx_cm.at[1], sem0),
        ]
        for k in range(K):
            copies.append(pltpu.async_copy(
                negflat_hbm.at[pl.ds(k * B + base, BW)],
                idx_cm.at[2 + k], sem0))
        for c in copies:
            c.wait()

        @pl.loop(0, NSUB)
        def _(s):
            for c in range(C):
                idx_g[pl.ds(s * R + c * L, L)] = idx_cm[c, pl.ds(s * L, L)]

        sems = (sem0, sem1)

        def start(s, par):
            pltpu.async_copy(emb_hbm.at[idx_g.at[pl.ds(s * R, R)]],
                             rows_v.at[pl.ds(par * R, R)], sems[par])

        def wait(par):
            pltpu.make_async_copy(emb_hbm.at[idx_g.at[pl.ds(0, R)]],
                                  rows_v.at[pl.ds(par * R, R)],
                                  sems[par]).wait()

        start(0, 0)

        lane = lax.iota(jnp.int32, L)
        zero = jnp.zeros((L,), jnp.float32)

        @pl.loop(0, NSUB)
        def _(j):
            for p in range(P):
                score_v[p, pl.ds(j * L, L)] = zero

        def compute(s, par):
            boff = par * R
            trow = lane + boff

            @pl.loop(0, DC)
            def _(dc):
                dbase = dc * L
                cvs = [dbase + ((lane + d) & (L - 1)) for d in range(L)]
                tcols = [plsc.load_gather(rows_v, [trow, cvs[d]])
                         for d in range(L)]
                for p in range(P):
                    prow = trow + (p + 1) * L
                    acc = tcols[0] * plsc.load_gather(rows_v, [prow, cvs[0]])
                    for d in range(1, L):
                        pc = plsc.load_gather(rows_v, [prow, cvs[d]])
                        acc = acc + tcols[d] * pc
                    plsc.addupdate(score_v.at[p, pl.ds(s * L, L)], acc)

        @pl.loop(0, NSUB)
        def _(s):
            par = s & 1
            npar = 1 - par

            @pl.when(s + 1 < NSUB)
            def _():
                @pl.when(npar == 0)
                def _():
                    start(s + 1, 0)

                @pl.when(npar == 1)
                def _():
                    start(s + 1, 1)

            @pl.when(par == 0)
            def _():
                wait(0)

            @pl.when(par == 1)
            def _():
                wait(1)

            compute(s, par)

        pltpu.sync_copy(score_v, out_hbm.at[wid])

    return sc_scores


def _tc_flatten(negT):
    M, Nc = negT.shape

    def body(i_ref, o_ref):
        o_ref[...] = i_ref[...].reshape(M * Nc // 128, 128)

    out = pl.pallas_call(
        body,
        out_shape=jax.ShapeDtypeStruct((M * Nc // 128, 128), jnp.int32),
    )(negT)
    return out.reshape(-1)


def _tc_loss(scores_flat, B, K, P):
    M, Nc = scores_flat.shape

    def body(s_ref, o_ref):
        s = s_ref[...]
        row = lax.broadcasted_iota(jnp.int32, (M, Nc), 0)
        is_pos = (row % P) == 0
        x = jnp.where(is_pos, s, -s)
        ls = jnp.log(1.0 / (1.0 + jnp.exp(-x)) + 1e-15)
        pos_sum = jnp.sum(jnp.where(is_pos, ls, 0.0))
        neg_sum = jnp.sum(ls) - pos_sum
        o_ref[0, 0] = -(pos_sum / B) - (neg_sum / (B * K))

    out = pl.pallas_call(
        body,
        out_shape=jax.ShapeDtypeStruct((1, 1), jnp.float32),
        out_specs=pl.BlockSpec(memory_space=pltpu.SMEM),
    )(scores_flat)
    return out[0, 0]


def kernel(target, context, neg_context, emb):
    B, = target.shape
    K = neg_context.shape[1]
    N, D = emb.shape
    P = K + 1

    negT = neg_context.T.astype(jnp.int32)
    negflat = _tc_flatten(negT)
    embT = emb.T
    tailT = lax.slice(embT, (0, N - 128), (D, N))
    emb_rm = _build_sc_transpose(N, D)(embT, tailT).reshape(N, D)
    scores = _build_sc_scores(B, K, D, N)(
        target.astype(jnp.int32), context.astype(jnp.int32), negflat, emb_rm)
    NW = scores.shape[0]
    return _tc_loss(scores.reshape(NW * P, B // NW), B, K, P)
